# Initial kernel scaffold; baseline (speedup 1.0000x reference)
import functools

import jax
import jax.numpy as jnp
from jax import lax
from jax.experimental import pallas as pl
from jax.experimental.pallas import tpu as pltpu
from jax.experimental.pallas import tpu_sc as plsc

H = W = 512
N = 2 * H * W
NQ = 8 * 3 * H * W
NW = 32
QPW = NQ // NW
CHUNK = 1024
NCHUNK = QPW // CHUNK
NVREG = CHUNK // 16
NSUB = CHUNK // 128
PRE = N // 8


def _sc_body(q_hbm, bnd_hbm, val_hbm, pre_hbm, out_hbm,
             pre_v, qbuf, lbuf, midb, cmpb, obuf, sbnd, sval, sem):
    cid = lax.axis_index("c")
    sid = lax.axis_index("s")
    wid = sid * 2 + cid

    pltpu.sync_copy(pre_hbm, pre_v)

    @pl.when(sid == 0)
    def _stage():
        pltpu.sync_copy(bnd_hbm, sbnd)
        pltpu.sync_copy(val_hbm, sval)

    plsc.subcore_barrier()

    def gather_round(table, dst):
        cps = []
        for k in range(NSUB):
            cps.append(pltpu.async_copy(
                table.at[midb.at[pl.ds(k * 128, 128)]],
                dst.at[pl.ds(k * 128, 128)], sem))
        for cp in cps:
            cp.wait()

    def pass_prefix(j, _):
        off = pl.multiple_of(j * 16, 16)
        q = qbuf[pl.ds(off, 16)]
        l = jnp.zeros((16,), jnp.int32)
        for d in range(16):
            i = (l >> 3) + jnp.int32(1 << (15 - d))
            t = plsc.load_gather(pre_v, [i])
            c = q > t
            l = l + jnp.where(c, jnp.int32(1 << (18 - d)), jnp.int32(0))
        lbuf[pl.ds(off, 16)] = l
        midb[pl.ds(off, 16)] = l + jnp.int32(4)
        return 0

    def make_mid_pass(step, nxt):
        def body(j, _):
            off = pl.multiple_of(j * 16, 16)
            q = qbuf[pl.ds(off, 16)]
            l = lbuf[pl.ds(off, 16)]
            t = cmpb[pl.ds(off, 16)]
            l = l + jnp.where(q > t, jnp.int32(step), jnp.int32(0))
            lbuf[pl.ds(off, 16)] = l
            midb[pl.ds(off, 16)] = l + jnp.int32(nxt)
            return 0
        return body

    def pass_final(j, _):
        off = pl.multiple_of(j * 16, 16)
        q = qbuf[pl.ds(off, 16)]
        l = lbuf[pl.ds(off, 16)]
        t = cmpb[pl.ds(off, 16)]
        res = l + (q > t).astype(jnp.int32)
        midb[pl.ds(off, 16)] = jnp.minimum(res, jnp.int32(N - 1))
        return 0

    def chunk_body(ch, _):
        base = pl.multiple_of(wid * QPW + ch * CHUNK, CHUNK)
        pltpu.sync_copy(q_hbm.at[pl.ds(base, CHUNK)], qbuf)
        lax.fori_loop(0, NVREG, pass_prefix, 0)
        gather_round(sbnd, cmpb)
        lax.fori_loop(0, NVREG, make_mid_pass(4, 2), 0)
        gather_round(sbnd, cmpb)
        lax.fori_loop(0, NVREG, make_mid_pass(2, 1), 0)
        gather_round(sbnd, cmpb)
        lax.fori_loop(0, NVREG, make_mid_pass(1, 0), 0)
        gather_round(sbnd, cmpb)
        lax.fori_loop(0, NVREG, pass_final, 0)
        gather_round(sval, obuf)
        pltpu.sync_copy(obuf, out_hbm.at[pl.ds(base, CHUNK)])
        return 0

    lax.fori_loop(0, NCHUNK, chunk_body, 0)


@jax.jit
def kernel(image, yx_res):
    b, c, h, w = yx_res.shape
    xs = (jnp.arange(w, dtype=jnp.float32) / (w - 1)) * 2.0 - 1.0
    ys = (jnp.arange(h, dtype=jnp.float32) / (h - 1)) * 2.0 - 1.0
    xm = jnp.broadcast_to(xs[None, :], (h, w))
    ym = jnp.broadcast_to(ys[:, None], (h, w))
    bnd = jnp.stack([xm + yx_res[0, 0], ym + yx_res[0, 1]], axis=-1).ravel()
    val = jnp.stack([xm + yx_res[1, 0], ym + yx_res[1, 1]], axis=-1).ravel()
    pre = bnd.reshape(PRE, 8)[:, 0]
    qflat = image.ravel()

    mesh = plsc.VectorSubcoreMesh(core_axis_name="c", subcore_axis_name="s")
    out = pl.kernel(
        _sc_body,
        out_type=jax.ShapeDtypeStruct((NQ,), jnp.float32),
        mesh=mesh,
        scratch_types=[
            pltpu.VMEM((PRE,), jnp.float32),
            pltpu.VMEM((CHUNK,), jnp.float32),
            pltpu.VMEM((CHUNK,), jnp.int32),
            pltpu.VMEM((CHUNK,), jnp.int32),
            pltpu.VMEM((CHUNK,), jnp.float32),
            pltpu.VMEM((CHUNK,), jnp.float32),
            pltpu.VMEM_SHARED((N,), jnp.float32),
            pltpu.VMEM_SHARED((N,), jnp.float32),
            pltpu.SemaphoreType.DMA,
        ],
    )(qflat, bnd, val, pre)
    return out.reshape(image.shape)


# speedup vs baseline: 174.4937x; 174.4937x over previous
import functools

import jax
import jax.numpy as jnp
from jax import lax
from jax.experimental import pallas as pl
from jax.experimental.pallas import tpu as pltpu
from jax.experimental.pallas import tpu_sc as plsc

H = W = 512
N = 2 * H * W
NQ = 8 * 3 * H * W
NW = 32
QPW = NQ // NW
CHUNK = 1024
NCHUNK = QPW // CHUNK
NVREG = CHUNK // 16
NSUB = CHUNK // 128
PRE = N // 8


def _sc_body(q_hbm, bnd_hbm, val_hbm, pre_hbm, out_hbm,
             pre_v, qbuf, lbuf, midb, cmpb, obuf, sbnd, sem):
    cid = lax.axis_index("c")
    sid = lax.axis_index("s")
    wid = sid * 2 + cid

    pltpu.sync_copy(pre_hbm, pre_v)

    @pl.when(sid == 0)
    def _stage():
        pltpu.sync_copy(bnd_hbm, sbnd)

    plsc.subcore_barrier()

    def gather_round(table, dst):
        cps = []
        for k in range(NSUB):
            cps.append(pltpu.async_copy(
                table.at[midb.at[pl.ds(k * 128, 128)]],
                dst.at[pl.ds(k * 128, 128)], sem))
        for cp in cps:
            cp.wait()

    def pass_prefix(j, _):
        off = pl.multiple_of(j * 16, 16)
        q = qbuf[pl.ds(off, 16)]
        l = jnp.zeros((16,), jnp.int32)
        for d in range(16):
            i = (l >> 3) + jnp.int32(1 << (15 - d))
            t = plsc.load_gather(pre_v, [i])
            c = q > t
            l = l + jnp.where(c, jnp.int32(1 << (18 - d)), jnp.int32(0))
        lbuf[pl.ds(off, 16)] = l
        midb[pl.ds(off, 16)] = l + jnp.int32(4)
        return 0

    def make_mid_pass(step, nxt):
        def body(j, _):
            off = pl.multiple_of(j * 16, 16)
            q = qbuf[pl.ds(off, 16)]
            l = lbuf[pl.ds(off, 16)]
            t = cmpb[pl.ds(off, 16)]
            l = l + jnp.where(q > t, jnp.int32(step), jnp.int32(0))
            lbuf[pl.ds(off, 16)] = l
            midb[pl.ds(off, 16)] = l + jnp.int32(nxt)
            return 0
        return body

    def pass_final(j, _):
        off = pl.multiple_of(j * 16, 16)
        q = qbuf[pl.ds(off, 16)]
        l = lbuf[pl.ds(off, 16)]
        t = cmpb[pl.ds(off, 16)]
        res = l + (q > t).astype(jnp.int32)
        midb[pl.ds(off, 16)] = jnp.minimum(res, jnp.int32(N - 1))
        return 0

    def chunk_body(ch, _):
        base = pl.multiple_of(wid * QPW + ch * CHUNK, CHUNK)
        pltpu.sync_copy(q_hbm.at[pl.ds(base, CHUNK)], qbuf)
        lax.fori_loop(0, NVREG, pass_prefix, 0)
        gather_round(sbnd, cmpb)
        lax.fori_loop(0, NVREG, make_mid_pass(4, 2), 0)
        gather_round(sbnd, cmpb)
        lax.fori_loop(0, NVREG, make_mid_pass(2, 1), 0)
        gather_round(sbnd, cmpb)
        lax.fori_loop(0, NVREG, make_mid_pass(1, 0), 0)
        gather_round(sbnd, cmpb)
        lax.fori_loop(0, NVREG, pass_final, 0)
        gather_round(val_hbm, obuf)
        pltpu.sync_copy(obuf, out_hbm.at[pl.ds(base, CHUNK)])
        return 0

    lax.fori_loop(0, NCHUNK, chunk_body, 0)


@jax.jit
def kernel(image, yx_res):
    b, c, h, w = yx_res.shape
    xs = (jnp.arange(w, dtype=jnp.float32) / (w - 1)) * 2.0 - 1.0
    ys = (jnp.arange(h, dtype=jnp.float32) / (h - 1)) * 2.0 - 1.0
    xm = jnp.broadcast_to(xs[None, :], (h, w))
    ym = jnp.broadcast_to(ys[:, None], (h, w))
    bnd = jnp.stack([xm + yx_res[0, 0], ym + yx_res[0, 1]], axis=-1).ravel()
    val = jnp.stack([xm + yx_res[1, 0], ym + yx_res[1, 1]], axis=-1).ravel()
    pre = bnd.reshape(PRE, 8)[:, 0]
    qflat = image.ravel()

    mesh = plsc.VectorSubcoreMesh(core_axis_name="c", subcore_axis_name="s")
    out = pl.kernel(
        _sc_body,
        out_type=jax.ShapeDtypeStruct((NQ,), jnp.float32),
        mesh=mesh,
        compiler_params=pltpu.CompilerParams(needs_layout_passes=False),
        scratch_types=[
            pltpu.VMEM((PRE,), jnp.float32),
            pltpu.VMEM((CHUNK,), jnp.float32),
            pltpu.VMEM((CHUNK,), jnp.int32),
            pltpu.VMEM((CHUNK,), jnp.int32),
            pltpu.VMEM((CHUNK,), jnp.float32),
            pltpu.VMEM((CHUNK,), jnp.float32),
            pltpu.VMEM_SHARED((N,), jnp.float32),
            pltpu.SemaphoreType.DMA,
        ],
    )(qflat, bnd, val, pre)
    return out.reshape(image.shape)
